# baseline (speedup 1.0000x reference)
import jax
import jax.numpy as jnp
from jax import lax
from jax.experimental import pallas as pl
from jax.experimental.pallas import tpu as pltpu

_B, _T, _IN = 8, 512, 80
_H, _K, _D = 512, 1024, 512
_BS = 4


def _mm(a, b):
    return lax.dot_general(a, b, (((1,), (0,)), ((), ())),
                           preferred_element_type=jnp.float32)


def _imcol(x):
    zrow = jnp.zeros((1, x.shape[1]), x.dtype)
    m_parts, p_parts = [], []
    for s in range(_BS):
        lo = s * _T
        m_parts += [zrow, x[lo:lo + _T - 1]]
        p_parts += [x[lo + 1:lo + _T], zrow]
    xm = jnp.concatenate(m_parts, axis=0)
    xp = jnp.concatenate(p_parts, axis=0)
    return jnp.concatenate([xm, x, xp], axis=1)


def _body(x_ref, w1_ref, b1_ref, w2_ref, b2_ref, ct_ref, csq_ref, cb_ref,
          w3_ref, b3_ref, w4_ref, b4_ref, idx_ref, out_ref):
    x = x_ref[...].reshape(_BS * _T, _IN)
    z1 = jnp.maximum(
        _mm(_imcol(x.astype(jnp.bfloat16)), w1_ref[...]) + b1_ref[...], 0.0)
    z2 = _mm(_imcol(z1.astype(jnp.bfloat16)), w2_ref[...]) + b2_ref[...]
    dot = _mm(z2.astype(jnp.bfloat16), ct_ref[...])
    d = csq_ref[...] - 2.0 * dot
    m = jnp.min(d, axis=1, keepdims=True)
    cols = lax.broadcasted_iota(jnp.int32, d.shape, 1)
    idx = jnp.min(jnp.where(d == m, cols, _K), axis=1).astype(jnp.int32)
    onehot = (cols == idx[:, None]).astype(jnp.bfloat16)
    quant = _mm(onehot, cb_ref[...])
    z3 = jnp.maximum(
        _mm(_imcol(quant.astype(jnp.bfloat16)), w3_ref[...]) + b3_ref[...],
        0.0)
    r = _mm(_imcol(z3.astype(jnp.bfloat16)), w4_ref[...]) + b4_ref[...]
    idx_ref[...] = idx.reshape(_BS, 1, _T)
    out_ref[...] = r.reshape(_BS, _T, _IN)


_TC_PARAMS = pltpu.CompilerParams(dimension_semantics=("arbitrary",))


def kernel(mels, W1, b1, W2, b2, codebook, W3, b3, W4, b4):
    bf = jnp.bfloat16
    w1k = jnp.transpose(W1.astype(bf), (2, 1, 0)).reshape(3 * _IN, _H)
    w2k = jnp.transpose(W2.astype(bf), (2, 1, 0)).reshape(3 * _H, _D)
    w3k = jnp.transpose(W3.astype(bf), (2, 1, 0)).reshape(3 * _D, _H)
    w4k = jnp.transpose(W4.astype(bf), (2, 1, 0)).reshape(3 * _H, _IN)
    ct = jnp.transpose(codebook)
    ctb = ct.astype(bf)
    csq = (ct ** 2).sum(0, keepdims=True)
    cbb = codebook.astype(bf)

    idx3, recon = pl.pallas_call(
        _body,
        grid=(_B // _BS,),
        in_specs=[
            pl.BlockSpec((_BS, _T, _IN), lambda b: (b, 0, 0)),
            pl.BlockSpec((3 * _IN, _H), lambda b: (0, 0)),
            pl.BlockSpec((1, _H), lambda b: (0, 0)),
            pl.BlockSpec((3 * _H, _D), lambda b: (0, 0)),
            pl.BlockSpec((1, _D), lambda b: (0, 0)),
            pl.BlockSpec((_D, _K), lambda b: (0, 0)),
            pl.BlockSpec((1, _K), lambda b: (0, 0)),
            pl.BlockSpec((_K, _D), lambda b: (0, 0)),
            pl.BlockSpec((3 * _D, _H), lambda b: (0, 0)),
            pl.BlockSpec((1, _H), lambda b: (0, 0)),
            pl.BlockSpec((3 * _H, _IN), lambda b: (0, 0)),
            pl.BlockSpec((1, _IN), lambda b: (0, 0)),
        ],
        out_specs=[
            pl.BlockSpec((_BS, 1, _T), lambda b: (b, 0, 0)),
            pl.BlockSpec((_BS, _T, _IN), lambda b: (b, 0, 0)),
        ],
        out_shape=[
            jax.ShapeDtypeStruct((_B, 1, _T), jnp.int32),
            jax.ShapeDtypeStruct((_B, _T, _IN), jnp.float32),
        ],
        compiler_params=_TC_PARAMS,
    )(mels, w1k, b1.reshape(1, _H), w2k, b2.reshape(1, _D), ctb, csq, cbb,
      w3k, b3.reshape(1, _H), w4k, b4.reshape(1, _IN))

    return (recon, idx3.reshape(_B, _T))


# speedup vs baseline: 1.0077x; 1.0077x over previous
import jax
import jax.numpy as jnp
from jax import lax
from jax.experimental import pallas as pl
from jax.experimental.pallas import tpu as pltpu

_B, _T, _IN = 8, 512, 80
_H, _K, _D = 512, 1024, 512
_BS = 4


def _mm(a, b):
    return lax.dot_general(a, b, (((1,), (0,)), ((), ())),
                           preferred_element_type=jnp.float32)


def _imcol(x):
    zrow = jnp.zeros((1, x.shape[1]), x.dtype)
    m_parts, p_parts = [], []
    for s in range(_BS):
        lo = s * _T
        m_parts += [zrow, x[lo:lo + _T - 1]]
        p_parts += [x[lo + 1:lo + _T], zrow]
    xm = jnp.concatenate(m_parts, axis=0)
    xp = jnp.concatenate(p_parts, axis=0)
    return jnp.concatenate([xm, x, xp], axis=1)


def _body(x_ref, w1_ref, b1_ref, w2_ref, b2_ref, ct_ref, csq_ref, cb_ref,
          w3_ref, b3_ref, w4_ref, b4_ref, idx_ref, out_ref):
    x = x_ref[...].reshape(_BS * _T, _IN)
    z1 = jnp.maximum(_mm(_imcol(x), w1_ref[...]) + b1_ref[...], 0.0)
    z2 = _mm(_imcol(z1), w2_ref[...]) + b2_ref[...]
    dot = _mm(z2, ct_ref[...])
    d = csq_ref[...] - 2.0 * dot
    m = jnp.min(d, axis=1, keepdims=True)
    cols = lax.broadcasted_iota(jnp.int32, d.shape, 1)
    idx = jnp.min(jnp.where(d == m, cols, _K), axis=1).astype(jnp.int32)
    onehot = (cols == idx[:, None]).astype(jnp.bfloat16)
    quant = _mm(onehot, cb_ref[...])
    z3 = jnp.maximum(
        _mm(_imcol(quant.astype(jnp.bfloat16)), w3_ref[...]) + b3_ref[...],
        0.0)
    r = _mm(_imcol(z3.astype(jnp.bfloat16)), w4_ref[...]) + b4_ref[...]
    idx_ref[...] = idx.reshape(_BS, 1, _T)
    out_ref[...] = r.reshape(_BS, _T, _IN)


_TC_PARAMS = pltpu.CompilerParams(dimension_semantics=("arbitrary",))


def kernel(mels, W1, b1, W2, b2, codebook, W3, b3, W4, b4):
    bf = jnp.bfloat16
    w1k = jnp.transpose(W1, (2, 1, 0)).reshape(3 * _IN, _H)
    w2k = jnp.transpose(W2, (2, 1, 0)).reshape(3 * _H, _D)
    w3k = jnp.transpose(W3.astype(bf), (2, 1, 0)).reshape(3 * _D, _H)
    w4k = jnp.transpose(W4.astype(bf), (2, 1, 0)).reshape(3 * _H, _IN)
    ct = jnp.transpose(codebook)
    csq = (ct ** 2).sum(0, keepdims=True)
    cbb = codebook.astype(bf)

    idx3, recon = pl.pallas_call(
        _body,
        grid=(_B // _BS,),
        in_specs=[
            pl.BlockSpec((_BS, _T, _IN), lambda b: (b, 0, 0)),
            pl.BlockSpec((3 * _IN, _H), lambda b: (0, 0)),
            pl.BlockSpec((1, _H), lambda b: (0, 0)),
            pl.BlockSpec((3 * _H, _D), lambda b: (0, 0)),
            pl.BlockSpec((1, _D), lambda b: (0, 0)),
            pl.BlockSpec((_D, _K), lambda b: (0, 0)),
            pl.BlockSpec((1, _K), lambda b: (0, 0)),
            pl.BlockSpec((_K, _D), lambda b: (0, 0)),
            pl.BlockSpec((3 * _D, _H), lambda b: (0, 0)),
            pl.BlockSpec((1, _H), lambda b: (0, 0)),
            pl.BlockSpec((3 * _H, _IN), lambda b: (0, 0)),
            pl.BlockSpec((1, _IN), lambda b: (0, 0)),
        ],
        out_specs=[
            pl.BlockSpec((_BS, 1, _T), lambda b: (b, 0, 0)),
            pl.BlockSpec((_BS, _T, _IN), lambda b: (b, 0, 0)),
        ],
        out_shape=[
            jax.ShapeDtypeStruct((_B, 1, _T), jnp.int32),
            jax.ShapeDtypeStruct((_B, _T, _IN), jnp.float32),
        ],
        compiler_params=_TC_PARAMS,
    )(mels, w1k, b1.reshape(1, _H), w2k, b2.reshape(1, _D), ct, csq, cbb,
      w3k, b3.reshape(1, _H), w4k, b4.reshape(1, _IN))

    return (recon, idx3.reshape(_B, _T))
